# baseline (speedup 1.0000x reference)
import jax
import jax.numpy as jnp
from jax import lax
from jax.experimental import pallas as pl
from jax.experimental.pallas import tpu as pltpu

_S = 30.0


def _ldam_body(x_ref, m_ref, t_ref, out_ref):
    i = pl.program_id(0)
    nb = pl.num_programs(0)
    br, c = x_ref.shape
    x = x_ref[...]
    t = t_ref[0, 0, :]
    mrow = m_ref[0, :]

    ones = jnp.ones((c, 1), jnp.float32)

    def msum(v):
        return jnp.dot(v, ones, preferred_element_type=jnp.float32)[:, 0]

    col = lax.broadcasted_iota(jnp.int32, (br, c), 1)
    tmask = col == t[:, None]
    p = msum(jnp.where(tmask, x, 0.0))
    bm = msum(jnp.where(tmask, mrow[None, :], 0.0))

    rmax = jnp.max(x, axis=1)
    expd = jnp.exp(_S * x - (_S * rmax)[:, None])
    e = msum(expd)
    t1 = msum(jnp.where(tmask, expd, 0.0))
    z = e - t1 + jnp.exp(_S * (p - bm - rmax))
    lossb = _S * rmax + jnp.log(z) - _S * (p - bm)

    part = (jnp.sum(lossb) * (1.0 / (br * nb)))[None, None]

    @pl.when(i == 0)
    def _init():
        out_ref[...] = jnp.zeros((1, 1), jnp.float32)

    out_ref[...] += part


def kernel(x, m_list, target):
    b, c = x.shape
    br = 4096
    nb = b // br
    t3 = target.astype(jnp.int32).reshape(nb, 1, br)
    m2 = m_list.reshape(1, c)
    out = pl.pallas_call(
        _ldam_body,
        grid=(nb,),
        in_specs=[
            pl.BlockSpec((br, c), lambda i: (i, 0)),
            pl.BlockSpec((1, c), lambda i: (0, 0)),
            pl.BlockSpec((1, 1, br), lambda i: (i, 0, 0)),
        ],
        out_specs=pl.BlockSpec((1, 1), lambda i: (0, 0)),
        out_shape=jax.ShapeDtypeStruct((1, 1), jnp.float32),
        compiler_params=pltpu.CompilerParams(
            dimension_semantics=("arbitrary",),
            vmem_limit_bytes=100 * 1024 * 1024,
        ),
    )(x, m2, t3)
    return out[0, 0]
